# baseline (speedup 1.0000x reference)
import functools

import jax
import jax.numpy as jnp
from jax import lax
from jax.experimental import pallas as pl
from jax.experimental.pallas import tpu as pltpu
from jax.experimental.pallas import tpu_sc as plsc

HIDDEN = 128
TABLE_ROWS = 24
B_TOTAL = 512 * 4 * 16 * 16
CHUNK_R = 256
LANES = 16


def _make_gather():
    info = plsc.get_sparse_core_info()
    nc, ns = info.num_cores, info.num_subcores
    nw = nc * ns
    b_per_w = B_TOTAL // nw
    n_chunks = b_per_w // CHUNK_R
    groups_per_chunk = CHUNK_R // LANES
    mesh = plsc.VectorSubcoreMesh(core_axis_name="c", subcore_axis_name="s")

    @functools.partial(
        pl.kernel,
        mesh=mesh,
        out_type=jax.ShapeDtypeStruct((B_TOTAL * HIDDEN,), jnp.float32),
        scratch_types=[
            pltpu.VMEM((TABLE_ROWS * HIDDEN,), jnp.float32),
            pltpu.VMEM((b_per_w,), jnp.int32),
            pltpu.VMEM((CHUNK_R * HIDDEN,), jnp.float32),
            pltpu.VMEM((CHUNK_R * HIDDEN,), jnp.float32),
            pltpu.SemaphoreType.DMA,
        ],
        compiler_params=pltpu.CompilerParams(needs_layout_passes=False),
    )
    def gather_kernel(x_hbm, pe_hbm, out_hbm, table_v, idx_v, rows_a, rows_b, sem_o):
        bufs = (rows_a, rows_b)
        wid = lax.axis_index("s") * nc + lax.axis_index("c")
        base = wid * b_per_w
        pltpu.sync_copy(pe_hbm, table_v)
        pltpu.sync_copy(x_hbm.at[pl.ds(base, b_per_w)], idx_v)

        lane = lax.iota(jnp.int32, LANES)

        def expand_chunk(s, slot):
            def group(g, carry):
                row16 = idx_v[pl.ds(s * CHUNK_R + g * LANES, LANES)]
                gidx = row16 * HIDDEN
                sidx = (g * LANES + lane) * HIDDEN
                buf = bufs[slot]
                for _ in range(HIDDEN):
                    vals = plsc.load_gather(table_v, [gidx])
                    plsc.store_scatter(buf, [sidx], vals)
                    gidx = gidx + 1
                    sidx = sidx + 1
                return carry

            lax.fori_loop(0, groups_per_chunk, group, 0)

        def o_copy(s, slot):
            return pltpu.make_async_copy(
                bufs[slot],
                out_hbm.at[pl.ds((base + s * CHUNK_R) * HIDDEN, CHUNK_R * HIDDEN)],
                sem_o)

        expand_chunk(0, 0)
        o_copy(0, 0).start()
        expand_chunk(1, 1)
        o_copy(1, 1).start()

        def body(j, carry):
            s0 = 2 + 2 * j
            o_copy(s0 - 2, 0).wait()
            expand_chunk(s0, 0)
            o_copy(s0, 0).start()
            o_copy(s0 - 1, 1).wait()
            expand_chunk(s0 + 1, 1)
            o_copy(s0 + 1, 1).start()
            return carry

        lax.fori_loop(0, (n_chunks - 2) // 2, body, 0)

        o_copy(n_chunks - 2, 0).wait()
        o_copy(n_chunks - 1, 1).wait()

    return gather_kernel


def kernel(x, pe):
    orig_shape = x.shape
    flat = x.reshape(B_TOTAL).astype(jnp.int32)
    out = _make_gather()(flat, pe.reshape(TABLE_ROWS * HIDDEN))
    return out.reshape(*orig_shape, HIDDEN)


# speedup vs baseline: 4.6864x; 4.6864x over previous
import functools

import jax
import jax.numpy as jnp
from jax import lax
from jax.experimental import pallas as pl
from jax.experimental.pallas import tpu as pltpu
from jax.experimental.pallas import tpu_sc as plsc

HIDDEN = 128
TABLE_ROWS = 24
B_TOTAL = 512 * 4 * 16 * 16
CHUNK_R = 256
LANES = 16


def _make_gather():
    info = plsc.get_sparse_core_info()
    nc, ns = info.num_cores, info.num_subcores
    nw = nc * ns
    b_per_w = B_TOTAL // nw
    n_chunks = b_per_w // CHUNK_R
    groups_per_chunk = CHUNK_R // LANES
    mesh = plsc.VectorSubcoreMesh(core_axis_name="c", subcore_axis_name="s")

    @functools.partial(
        pl.kernel,
        mesh=mesh,
        out_type=jax.ShapeDtypeStruct((B_TOTAL * HIDDEN,), jnp.float32),
        scratch_types=[
            pltpu.VMEM((TABLE_ROWS * HIDDEN,), jnp.float32),
            pltpu.VMEM((b_per_w,), jnp.int32),
            pltpu.VMEM((CHUNK_R * HIDDEN,), jnp.float32),
            pltpu.VMEM((CHUNK_R * HIDDEN,), jnp.float32),
            pltpu.SemaphoreType.DMA,
        ],
        compiler_params=pltpu.CompilerParams(needs_layout_passes=False),
    )
    def gather_kernel(x_hbm, pe_hbm, out_hbm, table_v, idx_v, rows_a, rows_b, sem_o):
        bufs = (rows_a, rows_b)
        wid = lax.axis_index("s") * nc + lax.axis_index("c")
        base = wid * b_per_w
        pltpu.sync_copy(pe_hbm, table_v)
        pltpu.sync_copy(x_hbm.at[pl.ds(base, b_per_w)], idx_v)

        lane = lax.iota(jnp.int32, LANES)

        def expand_chunk(s, slot):
            buf = bufs[slot]

            def group(g, carry):
                row16 = idx_v[pl.ds(s * CHUNK_R + g * LANES, LANES)]
                for l in range(LANES):
                    src = row16[l] * HIDDEN
                    dst = (g * LANES + l) * HIDDEN
                    for j in range(0, HIDDEN, LANES):
                        buf[pl.ds(dst + j, LANES)] = table_v[pl.ds(src + j, LANES)]
                return carry

            lax.fori_loop(0, groups_per_chunk, group, 0)

        def o_copy(s, slot):
            return pltpu.make_async_copy(
                bufs[slot],
                out_hbm.at[pl.ds((base + s * CHUNK_R) * HIDDEN, CHUNK_R * HIDDEN)],
                sem_o)

        expand_chunk(0, 0)
        o_copy(0, 0).start()
        expand_chunk(1, 1)
        o_copy(1, 1).start()

        def body(j, carry):
            s0 = 2 + 2 * j
            o_copy(s0 - 2, 0).wait()
            expand_chunk(s0, 0)
            o_copy(s0, 0).start()
            o_copy(s0 - 1, 1).wait()
            expand_chunk(s0 + 1, 1)
            o_copy(s0 + 1, 1).start()
            return carry

        lax.fori_loop(0, (n_chunks - 2) // 2, body, 0)

        o_copy(n_chunks - 2, 0).wait()
        o_copy(n_chunks - 1, 1).wait()

    return gather_kernel


def kernel(x, pe):
    orig_shape = x.shape
    flat = x.reshape(B_TOTAL).astype(jnp.int32)
    out = _make_gather()(flat, pe.reshape(TABLE_ROWS * HIDDEN))
    return out.reshape(*orig_shape, HIDDEN)


# speedup vs baseline: 11.2210x; 2.3944x over previous
import functools

import jax
import jax.numpy as jnp
from jax import lax
from jax.experimental import pallas as pl
from jax.experimental.pallas import tpu as pltpu
from jax.experimental.pallas import tpu_sc as plsc

HIDDEN = 128
TABLE_ROWS = 24
B_TOTAL = 512 * 4 * 16 * 16
CHUNK_R = 256
LANES = 16


def _make_gather():
    info = plsc.get_sparse_core_info()
    nc, ns = info.num_cores, info.num_subcores
    nw = nc * ns
    b_per_w = B_TOTAL // nw
    n_chunks = b_per_w // CHUNK_R
    groups_per_chunk = CHUNK_R // LANES
    mesh = plsc.VectorSubcoreMesh(core_axis_name="c", subcore_axis_name="s")

    @functools.partial(
        pl.kernel,
        mesh=mesh,
        out_type=jax.ShapeDtypeStruct((B_TOTAL * HIDDEN,), jnp.float32),
        scratch_types=[
            pltpu.VMEM((TABLE_ROWS * HIDDEN,), jnp.float32),
            pltpu.VMEM((b_per_w,), jnp.int32),
            pltpu.VMEM((CHUNK_R * HIDDEN,), jnp.float32),
            pltpu.VMEM((CHUNK_R * HIDDEN,), jnp.float32),
            pltpu.SemaphoreType.DMA,
        ],
        compiler_params=pltpu.CompilerParams(needs_layout_passes=False),
    )
    def gather_kernel(x_hbm, pe_hbm, out_hbm, table_v, idx_v, rows_a, rows_b, sem_o):
        bufs = (rows_a, rows_b)
        wid = lax.axis_index("s") * nc + lax.axis_index("c")
        base = wid * b_per_w
        pltpu.sync_copy(pe_hbm, table_v)
        pltpu.sync_copy(x_hbm.at[pl.ds(base, b_per_w)], idx_v)

        lane = lax.iota(jnp.int32, LANES)

        def expand_chunk(s, slot):
            buf = bufs[slot]

            @plsc.parallel_loop(0, groups_per_chunk)
            def group(g):
                row16 = idx_v[pl.ds(s * CHUNK_R + g * LANES, LANES)]
                for l in range(LANES):
                    src = row16[l] * HIDDEN
                    dst = (g * LANES + l) * HIDDEN
                    for j in range(0, HIDDEN, LANES):
                        buf[pl.ds(dst + j, LANES)] = table_v[pl.ds(src + j, LANES)]

        def o_copy(s, slot):
            return pltpu.make_async_copy(
                bufs[slot],
                out_hbm.at[pl.ds((base + s * CHUNK_R) * HIDDEN, CHUNK_R * HIDDEN)],
                sem_o)

        expand_chunk(0, 0)
        o_copy(0, 0).start()
        expand_chunk(1, 1)
        o_copy(1, 1).start()

        def body(j, carry):
            s0 = 2 + 2 * j
            o_copy(s0 - 2, 0).wait()
            expand_chunk(s0, 0)
            o_copy(s0, 0).start()
            o_copy(s0 - 1, 1).wait()
            expand_chunk(s0 + 1, 1)
            o_copy(s0 + 1, 1).start()
            return carry

        lax.fori_loop(0, (n_chunks - 2) // 2, body, 0)

        o_copy(n_chunks - 2, 0).wait()
        o_copy(n_chunks - 1, 1).wait()

    return gather_kernel


def kernel(x, pe):
    orig_shape = x.shape
    flat = x.reshape(B_TOTAL).astype(jnp.int32)
    out = _make_gather()(flat, pe.reshape(TABLE_ROWS * HIDDEN))
    return out.reshape(*orig_shape, HIDDEN)
